# Initial kernel scaffold; baseline (speedup 1.0000x reference)
import jax
import jax.numpy as jnp
from jax.experimental import pallas as pl


def kernel(x, token_counts, total_tokens):
    raise NotImplementedError("write your pallas kernel here")


# speedup vs baseline: 134.2799x; 134.2799x over previous
import functools

import jax
import jax.numpy as jnp
from jax import lax
from jax.experimental import pallas as pl
from jax.experimental.pallas import tpu as pltpu
from jax.experimental.pallas import tpu_sc as plsc

_VOCAB = 1_000_000
_VOCAB_PAD = 1_048_576
_TROWS = 8192
_TGRID = 8

_B, _N = 16384, 200
_TOTAL = _B * _N
_NC, _NS = 2, 16
_NW = _NC * _NS
_LANES = 128
_PER_W = _TOTAL // _NW
_CHUNK = 12_800
_NCHUNKS = _PER_W // _CHUNK


def _table_body(total_ref, counts_ref, out_ref):
    total = total_ref[0, 0]
    freq = counts_ref[...] / total
    v = -jnp.log(freq + 1e-9)
    out_ref[...] = jax.nn.sigmoid(v - 5.0)


def _transform_table(counts_padded, total_tokens):
    blk = _TROWS // _TGRID
    return pl.pallas_call(
        _table_body,
        grid=(_TGRID,),
        in_specs=[
            pl.BlockSpec(memory_space=pltpu.SMEM),
            pl.BlockSpec((blk, _LANES), lambda i: (i, 0)),
        ],
        out_specs=pl.BlockSpec((blk, _LANES), lambda i: (i, 0)),
        out_shape=jax.ShapeDtypeStruct((_TROWS, _LANES), jnp.float32),
    )(jnp.reshape(total_tokens, (1, 1)), counts_padded)


def _gather_body(x_hbm, table_hbm, out_hbm, idx_v, val_v, sem):
    wid = lax.axis_index("s") * _NC + lax.axis_index("c")
    base = wid * _PER_W
    for c in range(_NCHUNKS):
        off = base + c * _CHUNK
        pltpu.sync_copy(x_hbm.at[pl.ds(off, _CHUNK)], idx_v)
        pltpu.async_copy(table_hbm.at[idx_v], val_v, sem).wait()
        pltpu.sync_copy(val_v, out_hbm.at[pl.ds(off, _CHUNK)])


_gather = pl.kernel(
    _gather_body,
    out_type=jax.ShapeDtypeStruct((_TOTAL,), jnp.float32),
    mesh=plsc.VectorSubcoreMesh(core_axis_name="c", subcore_axis_name="s"),
    scratch_types=[
        pltpu.VMEM((_CHUNK,), jnp.int32),
        pltpu.VMEM((_CHUNK,), jnp.float32),
        pltpu.SemaphoreType.DMA,
    ],
)


@jax.jit
def kernel(x, token_counts, total_tokens):
    counts_padded = jnp.concatenate(
        [token_counts, jnp.ones((_VOCAB_PAD - _VOCAB,), jnp.float32)]
    ).reshape(_TROWS, _LANES)
    table = _transform_table(counts_padded, total_tokens).reshape(-1)
    out = _gather(x.reshape(-1), table)
    return out.reshape(_B, _N)


# speedup vs baseline: 211.1453x; 1.5724x over previous
import functools

import jax
import jax.numpy as jnp
from jax import lax
from jax.experimental import pallas as pl
from jax.experimental.pallas import tpu as pltpu
from jax.experimental.pallas import tpu_sc as plsc

_VOCAB = 1_000_000
_VOCAB_PAD = 1_048_576
_TROWS = 8192
_TGRID = 8

_B, _N = 16384, 200
_TOTAL = _B * _N
_NC, _NS = 2, 16
_NW = _NC * _NS
_LANES = 128
_PER_W = _TOTAL // _NW
_CHUNK = 12_800
_NCHUNKS = _PER_W // _CHUNK


def _table_body(total_ref, counts_ref, out_ref):
    total = total_ref[0, 0]
    freq = counts_ref[...] / total
    v = -jnp.log(freq + 1e-9)
    out_ref[...] = jax.nn.sigmoid(v - 5.0)


def _transform_table(counts_padded, total_tokens):
    blk = _TROWS // _TGRID
    return pl.pallas_call(
        _table_body,
        grid=(_TGRID,),
        in_specs=[
            pl.BlockSpec(memory_space=pltpu.SMEM),
            pl.BlockSpec((blk, _LANES), lambda i: (i, 0)),
        ],
        out_specs=pl.BlockSpec((blk, _LANES), lambda i: (i, 0)),
        out_shape=jax.ShapeDtypeStruct((_TROWS, _LANES), jnp.float32),
    )(jnp.reshape(total_tokens, (1, 1)), counts_padded)


def _gather_body(x_hbm, table_hbm, out_hbm, tab_s, idx_v, val_v, sem):
    sid = lax.axis_index("s")
    wid = sid * _NC + lax.axis_index("c")

    @pl.when(sid == 0)
    def _():
        pltpu.sync_copy(table_hbm, tab_s)

    plsc.subcore_barrier()

    base = wid * _PER_W
    for c in range(_NCHUNKS):
        off = base + c * _CHUNK
        pltpu.sync_copy(x_hbm.at[pl.ds(off, _CHUNK)], idx_v)
        pltpu.async_copy(tab_s.at[idx_v], val_v, sem).wait()
        pltpu.sync_copy(val_v, out_hbm.at[pl.ds(off, _CHUNK)])


_gather = pl.kernel(
    _gather_body,
    out_type=jax.ShapeDtypeStruct((_TOTAL,), jnp.float32),
    mesh=plsc.VectorSubcoreMesh(core_axis_name="c", subcore_axis_name="s"),
    scratch_types=[
        pltpu.VMEM_SHARED((_VOCAB_PAD,), jnp.float32),
        pltpu.VMEM((_CHUNK,), jnp.int32),
        pltpu.VMEM((_CHUNK,), jnp.float32),
        pltpu.SemaphoreType.DMA,
    ],
)


@jax.jit
def kernel(x, token_counts, total_tokens):
    counts_padded = jnp.concatenate(
        [token_counts, jnp.ones((_VOCAB_PAD - _VOCAB,), jnp.float32)]
    ).reshape(_TROWS, _LANES)
    table = _transform_table(counts_padded, total_tokens).reshape(-1)
    out = _gather(x.reshape(-1), table)
    return out.reshape(_B, _N)


# speedup vs baseline: 230.3372x; 1.0909x over previous
import functools

import jax
import jax.numpy as jnp
from jax import lax
from jax.experimental import pallas as pl
from jax.experimental.pallas import tpu as pltpu
from jax.experimental.pallas import tpu_sc as plsc

_VOCAB = 1_000_000
_VOCAB_PAD = 1_048_576
_TROWS = 8192
_TGRID = 8

_B, _N = 16384, 200
_TOTAL = _B * _N
_NC, _NS = 2, 16
_NW = _NC * _NS
_LANES = 128
_PER_W = _TOTAL // _NW
_CHUNK = 12_800
_NCHUNKS = _PER_W // _CHUNK


def _table_body(total_ref, counts_ref, out_ref):
    total = total_ref[0, 0]
    freq = counts_ref[...] / total
    v = -jnp.log(freq + 1e-9)
    out_ref[...] = jax.nn.sigmoid(v - 5.0)


def _transform_table(counts_padded, total_tokens):
    blk = _TROWS // _TGRID
    return pl.pallas_call(
        _table_body,
        grid=(_TGRID,),
        in_specs=[
            pl.BlockSpec(memory_space=pltpu.SMEM),
            pl.BlockSpec((blk, _LANES), lambda i: (i, 0)),
        ],
        out_specs=pl.BlockSpec((blk, _LANES), lambda i: (i, 0)),
        out_shape=jax.ShapeDtypeStruct((_TROWS, _LANES), jnp.float32),
    )(jnp.reshape(total_tokens, (1, 1)), counts_padded)


def _gather_body(x_hbm, table_hbm, out_hbm, tab_s,
                 idx0, idx1, val0, val1,
                 tsem, isem0, isem1, gsem, osem0, osem1):
    sid = lax.axis_index("s")
    wid = sid * _NC + lax.axis_index("c")
    base = wid * _PER_W

    idx_bufs, val_bufs = [idx0, idx1], [val0, val1]
    isems, osems = [isem0, isem1], [osem0, osem1]

    @pl.when(sid == 0)
    def _():
        pltpu.async_copy(table_hbm, tab_s, tsem).wait()

    idx_loads = [
        pltpu.make_async_copy(
            x_hbm.at[pl.ds(base + c * _CHUNK, _CHUNK)], idx_bufs[c % 2],
            isems[c % 2])
        for c in range(_NCHUNKS)
    ]
    out_stores = [
        pltpu.make_async_copy(
            val_bufs[c % 2], out_hbm.at[pl.ds(base + c * _CHUNK, _CHUNK)],
            osems[c % 2])
        for c in range(_NCHUNKS)
    ]

    idx_loads[0].start()
    plsc.subcore_barrier()

    for c in range(_NCHUNKS):
        b = c % 2
        if c >= 2:
            out_stores[c - 2].wait()
        idx_loads[c].wait()
        if c + 1 < _NCHUNKS:
            idx_loads[c + 1].start()
        pltpu.async_copy(tab_s.at[idx_bufs[b]], val_bufs[b], gsem).wait()
        out_stores[c].start()

    out_stores[_NCHUNKS - 2].wait()
    out_stores[_NCHUNKS - 1].wait()


_gather = pl.kernel(
    _gather_body,
    out_type=jax.ShapeDtypeStruct((_TOTAL,), jnp.float32),
    mesh=plsc.VectorSubcoreMesh(core_axis_name="c", subcore_axis_name="s"),
    scratch_types=[
        pltpu.VMEM_SHARED((_VOCAB_PAD,), jnp.float32),
        pltpu.VMEM((_CHUNK,), jnp.int32),
        pltpu.VMEM((_CHUNK,), jnp.int32),
        pltpu.VMEM((_CHUNK,), jnp.float32),
        pltpu.VMEM((_CHUNK,), jnp.float32),
        pltpu.SemaphoreType.DMA,
        pltpu.SemaphoreType.DMA,
        pltpu.SemaphoreType.DMA,
        pltpu.SemaphoreType.DMA,
        pltpu.SemaphoreType.DMA,
        pltpu.SemaphoreType.DMA,
    ],
)


@jax.jit
def kernel(x, token_counts, total_tokens):
    counts_padded = jnp.concatenate(
        [token_counts, jnp.ones((_VOCAB_PAD - _VOCAB,), jnp.float32)]
    ).reshape(_TROWS, _LANES)
    table = _transform_table(counts_padded, total_tokens).reshape(-1)
    out = _gather(x.reshape(-1), table)
    return out.reshape(_B, _N)


# speedup vs baseline: 230.5537x; 1.0009x over previous
import functools

import jax
import jax.numpy as jnp
import numpy as np
from jax import lax
from jax.experimental import pallas as pl
from jax.experimental.pallas import tpu as pltpu
from jax.experimental.pallas import tpu_sc as plsc

_VOCAB = 1_000_000
_VOCAB_PAD = 1_048_576
_TROWS = 8192
_TGRID = 8

_B, _N = 16384, 200
_TOTAL = _B * _N
_NC, _NS = 2, 16
_NW = _NC * _NS
_LANES = 128
_PER_W = _TOTAL // _NW
_CROWS = 64
_CHUNK = _CROWS * _N
_ROWS_PER_W = _B // _NW
_NCHUNKS = _ROWS_PER_W // _CROWS


_E5 = float(np.exp(5.0))


def _table_body(total_ref, counts_ref, out_ref):
    total = total_ref[0, 0]
    freq = counts_ref[...] / total
    out_ref[...] = 1.0 / (1.0 + (freq + 1e-9) * _E5)


def _transform_table(counts_padded, total_tokens):
    blk = _TROWS // _TGRID
    return pl.pallas_call(
        _table_body,
        grid=(_TGRID,),
        in_specs=[
            pl.BlockSpec(memory_space=pltpu.SMEM),
            pl.BlockSpec((blk, _LANES), lambda i: (i, 0)),
        ],
        out_specs=pl.BlockSpec((blk, _LANES), lambda i: (i, 0)),
        out_shape=jax.ShapeDtypeStruct((_TROWS, _LANES), jnp.float32),
    )(jnp.reshape(total_tokens, (1, 1)), counts_padded)


def _gather_body(x_hbm, table_hbm, out_hbm, tab_s,
                 idx0, idx1, val0, val1,
                 tsem, isem0, isem1, gsem, osem0, osem1):
    sid = lax.axis_index("s")
    wid = sid * _NC + lax.axis_index("c")
    base = wid * _PER_W

    idx_bufs, val_bufs = [idx0, idx1], [val0, val1]
    isems, osems = [isem0, isem1], [osem0, osem1]

    @pl.when(sid == 0)
    def _():
        pltpu.async_copy(table_hbm, tab_s, tsem).wait()

    idx_loads = [
        pltpu.make_async_copy(
            x_hbm.at[pl.ds(base + c * _CHUNK, _CHUNK)], idx_bufs[c % 2],
            isems[c % 2])
        for c in range(_NCHUNKS)
    ]
    out_stores = [
        pltpu.make_async_copy(
            val_bufs[c % 2], out_hbm.at[pl.ds(base + c * _CHUNK, _CHUNK)],
            osems[c % 2])
        for c in range(_NCHUNKS)
    ]

    idx_loads[0].start()
    plsc.subcore_barrier()

    for c in range(_NCHUNKS):
        b = c % 2
        if c >= 2:
            out_stores[c - 2].wait()
        idx_loads[c].wait()
        if c + 1 < _NCHUNKS:
            idx_loads[c + 1].start()
        pltpu.async_copy(tab_s.at[idx_bufs[b]], val_bufs[b], gsem).wait()
        out_stores[c].start()

    out_stores[_NCHUNKS - 2].wait()
    out_stores[_NCHUNKS - 1].wait()


_gather = pl.kernel(
    _gather_body,
    out_type=jax.ShapeDtypeStruct((_TOTAL,), jnp.float32),
    mesh=plsc.VectorSubcoreMesh(core_axis_name="c", subcore_axis_name="s"),
    scratch_types=[
        pltpu.VMEM_SHARED((_VOCAB_PAD,), jnp.float32),
        pltpu.VMEM((_CHUNK,), jnp.int32),
        pltpu.VMEM((_CHUNK,), jnp.int32),
        pltpu.VMEM((_CHUNK,), jnp.float32),
        pltpu.VMEM((_CHUNK,), jnp.float32),
        pltpu.SemaphoreType.DMA,
        pltpu.SemaphoreType.DMA,
        pltpu.SemaphoreType.DMA,
        pltpu.SemaphoreType.DMA,
        pltpu.SemaphoreType.DMA,
        pltpu.SemaphoreType.DMA,
    ],
)


@jax.jit
def kernel(x, token_counts, total_tokens):
    counts_padded = jnp.concatenate(
        [token_counts, jnp.ones((_VOCAB_PAD - _VOCAB,), jnp.float32)]
    ).reshape(_TROWS, _LANES)
    table = _transform_table(counts_padded, total_tokens).reshape(-1)
    out = _gather(x.reshape(-1), table)
    return out.reshape(_B, _N)
